# baseline (speedup 1.0000x reference)
import jax
import jax.numpy as jnp
from jax import lax
from jax.experimental import pallas as pl
from jax.experimental.pallas import tpu as pltpu
from jax.experimental.pallas import tpu_sc as plsc

N = 10000
E = 320000
F = 128
DE = 16
H = 64
G = 128

NCORES = 2
NSUB = 16
FH = F // 2
ECT = E // NSUB
CH = 40
IB = 2000
IBC = IB // CH
NBI = ECT // IB
RPT = 624
NREM = N - NSUB * RPT

BN = 400
NBLK = N // BN
BE = 2000
EBLK = E // BE


def _lrelu(v):
    return jnp.where(v >= 0, v, 0.01 * v)


def _edgefeat_body(ea_ref, w1_ref, b1_ref, w2_ref, b2_ref, c1_ref, c2_ref):
    ea = ea_ref[...]
    c1_ref[...] = jnp.dot(ea, w1_ref[...],
                          preferred_element_type=jnp.float32) + b1_ref[...]
    c2_ref[...] = jnp.dot(ea, w2_ref[...],
                          preferred_element_type=jnp.float32) + b2_ref[...]


def _edge_features(edge_attr, w_e1, b_e1, w_e2, b_e2):
    return pl.pallas_call(
        _edgefeat_body,
        grid=(EBLK, NCORES),
        in_specs=[
            pl.BlockSpec((BE, DE), lambda i, g: (i, 0)),
            pl.BlockSpec((DE, F), lambda i, g: (0, g)),
            pl.BlockSpec((1, F), lambda i, g: (0, g)),
            pl.BlockSpec((DE, F), lambda i, g: (0, g)),
            pl.BlockSpec((1, F), lambda i, g: (0, g)),
        ],
        out_specs=[
            pl.BlockSpec((BE, F), lambda i, g: (g * EBLK + i, 0)),
            pl.BlockSpec((BE, F), lambda i, g: (g * EBLK + i, 0)),
        ],
        out_shape=[
            jax.ShapeDtypeStruct((NCORES * E, F), jnp.float32),
            jax.ShapeDtypeStruct((NCORES * E, F), jnp.float32),
        ],
    )(edge_attr, w_e1, b_e1, w_e2, b_e2)


def _proj1_body(x_ref, wpd_ref, wps_ref, pd_ref, ps_ref):
    h = x_ref[...]
    pd_ref[...] = jnp.dot(h, wpd_ref[...], preferred_element_type=jnp.float32)
    ps_ref[...] = jnp.dot(h, wps_ref[...], preferred_element_type=jnp.float32)


def _proj1(x, wpd, wps):
    return pl.pallas_call(
        _proj1_body,
        grid=(NBLK, NCORES),
        in_specs=[
            pl.BlockSpec((BN, F), lambda i, g: (i, 0)),
            pl.BlockSpec((F, F), lambda i, g: (0, g)),
            pl.BlockSpec((F, F), lambda i, g: (0, g)),
        ],
        out_specs=[
            pl.BlockSpec((BN, F), lambda i, g: (g * NBLK + i, 0)),
            pl.BlockSpec((BN, F), lambda i, g: (g * NBLK + i, 0)),
        ],
        out_shape=[
            jax.ShapeDtypeStruct((NCORES * N, F), jnp.float32),
            jax.ShapeDtypeStruct((NCORES * N, F), jnp.float32),
        ],
    )(x, wpd, wps)


def _proj2_body(x_ref, p0_ref, p1_ref, wpd_ref, wps_ref, h_ref, pd_ref, ps_ref):
    h = x_ref[...] + jnp.concatenate([p0_ref[..., :FH], p1_ref[..., :FH]],
                                     axis=1)
    h_ref[...] = h
    pd_ref[...] = jnp.dot(h, wpd_ref[...], preferred_element_type=jnp.float32)
    ps_ref[...] = jnp.dot(h, wps_ref[...], preferred_element_type=jnp.float32)


def _proj2(x, partials, wpd, wps):
    return pl.pallas_call(
        _proj2_body,
        grid=(NBLK, NCORES),
        in_specs=[
            pl.BlockSpec((BN, F), lambda i, g: (i, 0)),
            pl.BlockSpec((BN, F), lambda i, g: (i, 0)),
            pl.BlockSpec((BN, F), lambda i, g: (i + NBLK, 0)),
            pl.BlockSpec((F, F), lambda i, g: (0, g)),
            pl.BlockSpec((F, F), lambda i, g: (0, g)),
        ],
        out_specs=[
            pl.BlockSpec((BN, F), lambda i, g: (i, 0)),
            pl.BlockSpec((BN, F), lambda i, g: (g * NBLK + i, 0)),
            pl.BlockSpec((BN, F), lambda i, g: (g * NBLK + i, 0)),
        ],
        out_shape=[
            jax.ShapeDtypeStruct((N, F), jnp.float32),
            jax.ShapeDtypeStruct((NCORES * N, F), jnp.float32),
            jax.ShapeDtypeStruct((NCORES * N, F), jnp.float32),
        ],
    )(x, partials, partials, wpd, wps)


_L2E = 1.4426950408889634
_MAGIC = 12582912.0


def _exp2_16(y):
    t = y + _MAGIC
    f = y - (t - _MAGIC)
    kbits = lax.bitcast_convert_type(t, jnp.int32) - 0x4B3FFF81
    scale = lax.bitcast_convert_type(kbits << 23, jnp.float32)
    p = 0.99992448 + f * (0.69312103 + f * (0.24264008 + f * 0.055922036))
    return scale * p


def _rcp16(b):
    r = lax.bitcast_convert_type(0x7EF311C3 - lax.bitcast_convert_type(b, jnp.int32),
                                 jnp.float32)
    r = r * (2.0 - b * r)
    r = r * (2.0 - b * r)
    return r


def _log1p16(w):
    return 9.9750326e-06 + w * (0.99923548 + w * (-0.49023072 + w * (0.28527268
        + w * (-0.13158183 + w * 0.030449005))))


def _gated_msg16(uf, us):
    yf = jnp.maximum(jnp.minimum(uf * -_L2E, 126.0), -126.0)
    ef = _exp2_16(yf)
    g = _rcp16(1.0 + ef)
    ys = jnp.maximum(jnp.abs(us) * -_L2E, -126.0)
    w = _exp2_16(ys)
    sp = jnp.maximum(us, 0.0) + _log1p16(w)
    return g * sp


def _sc_edge_body(pd_hbm, ps_hbm, c_hbm, dstg_hbm, srcg_hbm, dstr_hbm,
                  zeros_hbm, out_hbm,
                  dstr_v, dstg_v, srcg_v, dstc_v, pd0_v, pd1_v, ps0_v, ps1_v,
                  c0_v, c1_v, msg_v, acc_sh, sem0, sem1):
    cid = lax.axis_index("c")
    sid = lax.axis_index("s")
    noff = cid * N
    ebase = sid * ECT
    goff = cid * E + ebase
    rbase = sid * RPT
    xbase = NSUB * RPT + sid * 8

    pltpu.sync_copy(zeros_hbm.at[pl.ds(rbase, RPT)],
                    acc_sh.at[pl.ds(rbase, RPT)])

    @pl.when(sid < NREM // 8)
    def _():
        pltpu.sync_copy(zeros_hbm.at[pl.ds(xbase, 8)],
                        acc_sh.at[pl.ds(xbase, 8)])

    def zm(e, cc):
        for jj in range(FH // 16, F // 16):
            msg_v[e, pl.ds(16 * jj, 16)] = jnp.zeros((16,), jnp.float32)
        return cc

    lax.fori_loop(0, CH, zm, 0)
    plsc.subcore_barrier()

    pdb = (pd0_v, pd1_v)
    psb = (ps0_v, ps1_v)
    cb = (c0_v, c1_v)
    semb = (sem0, sem1)

    def block(b, carry):
        off = ebase + b * IB
        go = goff + b * IB
        pltpu.sync_copy(dstr_hbm.at[pl.ds(off, IB)], dstr_v)
        pltpu.sync_copy(dstg_hbm.at[pl.ds(go, IB)], dstg_v)
        pltpu.sync_copy(srcg_hbm.at[pl.ds(go, IB)], srcg_v)

        def issue(j, p):
            return (
                pltpu.async_copy(pd_hbm.at[dstg_v.at[pl.ds(j * CH, CH)]],
                                 pdb[p], semb[p]),
                pltpu.async_copy(ps_hbm.at[srcg_v.at[pl.ds(j * CH, CH)]],
                                 psb[p], semb[p]),
                pltpu.async_copy(c_hbm.at[pl.ds(go + j * CH, CH)], cb[p],
                                 semb[p]),
            )

        def consume(j, ds, p):
            for qo in (0, 16, CH - 16):
                dstc_v[pl.ds(qo, 16)] = dstr_v[pl.ds(j * CH + qo, 16)]
            for d in ds:
                d.wait()

            @plsc.parallel_loop(0, CH, 1, unroll=4)
            def edge(e):
                for jj in range(FH // 16):
                    lo = 16 * jj
                    uf = pdb[p][e, pl.ds(lo, 16)] + psb[p][e, pl.ds(lo, 16)] \
                        + cb[p][e, pl.ds(lo, 16)]
                    us = pdb[p][e, pl.ds(FH + lo, 16)] \
                        + psb[p][e, pl.ds(FH + lo, 16)] \
                        + cb[p][e, pl.ds(FH + lo, 16)]
                    msg_v[e, pl.ds(lo, 16)] = _gated_msg16(uf, us)

            pltpu.sync_copy(msg_v, acc_sh.at[dstc_v], add=True)

        def pair(t, cc):
            a = 2 * t
            da = issue(a, 0)
            db = issue(a + 1, 1)
            consume(a, da, 0)
            consume(a + 1, db, 1)
            return cc

        lax.fori_loop(0, IBC // 2, pair, 0)
        return carry

    lax.fori_loop(0, NBI, block, 0)
    plsc.subcore_barrier()
    pltpu.sync_copy(acc_sh.at[pl.ds(rbase, RPT)],
                    out_hbm.at[pl.ds(noff + rbase, RPT)])

    @pl.when(sid < NREM // 8)
    def _():
        pltpu.sync_copy(acc_sh.at[pl.ds(xbase, 8)],
                        out_hbm.at[pl.ds(noff + xbase, 8)])


_sc_edge = pl.kernel(
    _sc_edge_body,
    out_type=jax.ShapeDtypeStruct((NCORES * N, F), jnp.float32),
    mesh=plsc.VectorSubcoreMesh(core_axis_name="c", subcore_axis_name="s"),
    scratch_types=(
        [pltpu.VMEM((IB,), jnp.int32)] * 3
        + [pltpu.VMEM((CH,), jnp.int32)]
        + [pltpu.VMEM((CH, F), jnp.float32)] * 7
        + [
            pltpu.VMEM_SHARED((N, F), jnp.float32),
            pltpu.SemaphoreType.DMA,
            pltpu.SemaphoreType.DMA,
        ]
    ),
)


def _tail_body(h1_ref, p0_ref, p1_ref, batch_ref, w1_ref, b1_ref, w2_ref,
               b2_ref, wout_ref, bout_ref, out_ref, gacc):
    i = pl.program_id(0)

    @pl.when(i == 0)
    def _():
        gacc[...] = jnp.zeros_like(gacc)

    h2 = h1_ref[...] + jnp.concatenate([p0_ref[..., :FH], p1_ref[..., :FH]],
                                       axis=1)
    a = _lrelu(h2)
    t = jnp.dot(a, w1_ref[...], preferred_element_type=jnp.float32)
    t = _lrelu(t + b1_ref[...])
    gid = batch_ref[0, 0, :]
    cols = lax.broadcasted_iota(jnp.int32, (BN, G), 1)
    oh = (gid[:, None] == cols).astype(jnp.float32)
    gacc[...] += lax.dot_general(oh, t, (((0,), (0,)), ((), ())),
                                 preferred_element_type=jnp.float32)

    @pl.when(i == NBLK - 1)
    def _():
        g = gacc[...]
        z = jnp.dot(g, w2_ref[...], preferred_element_type=jnp.float32)
        z = _lrelu(z + b2_ref[...])
        o = jnp.sum(z * wout_ref[...], axis=1, keepdims=True) + bout_ref[...]
        out_ref[...] = jnp.maximum(o, 0.0)


def _tail(h1, partials, batch3d, w1, b1, w2, b2, wout_row, bout11):
    return pl.pallas_call(
        _tail_body,
        grid=(NBLK,),
        in_specs=[
            pl.BlockSpec((BN, F), lambda i: (i, 0)),
            pl.BlockSpec((BN, F), lambda i: (i, 0)),
            pl.BlockSpec((BN, F), lambda i: (i + NBLK, 0)),
            pl.BlockSpec((1, 1, BN), lambda i: (i, 0, 0)),
            pl.BlockSpec((F, H), lambda i: (0, 0)),
            pl.BlockSpec((1, H), lambda i: (0, 0)),
            pl.BlockSpec((H, H), lambda i: (0, 0)),
            pl.BlockSpec((1, H), lambda i: (0, 0)),
            pl.BlockSpec((1, H), lambda i: (0, 0)),
            pl.BlockSpec((1, 1), lambda i: (0, 0)),
        ],
        out_specs=pl.BlockSpec((G, 1), lambda i: (0, 0)),
        out_shape=jax.ShapeDtypeStruct((G, 1), jnp.float32),
        scratch_shapes=[pltpu.VMEM((G, H), jnp.float32)],
    )(h1, partials, partials, batch3d, w1, b1, w2, b2, wout_row, bout11)


def kernel(x, edge_attr, Wf1, bf1, Ws1, bs1, Wf2, bf2, Ws2, bs2,
           W1, b1, W2, b2, Wout, bout, edge_index, batch):
    f32 = jnp.float32
    src = edge_index[0].astype(jnp.int32)
    dst = edge_index[1].astype(jnp.int32)

    def corecat(wf, ws):
        return jnp.concatenate([wf[:, :FH], ws[:, :FH], wf[:, FH:], ws[:, FH:]], 1)

    wpd1 = corecat(Wf1[:F], Ws1[:F])
    wps1 = corecat(Wf1[F:2 * F], Ws1[F:2 * F])
    wpd2 = corecat(Wf2[:F], Ws2[:F])
    wps2 = corecat(Wf2[F:2 * F], Ws2[F:2 * F])
    w_e1 = corecat(Wf1[2 * F:], Ws1[2 * F:])
    w_e2 = corecat(Wf2[2 * F:], Ws2[2 * F:])
    b_e1 = corecat(bf1.reshape(1, F), bs1.reshape(1, F))
    b_e2 = corecat(bf2.reshape(1, F), bs2.reshape(1, F))

    c1, c2 = _edge_features(edge_attr.astype(f32), w_e1, b_e1, w_e2, b_e2)
    zeros_n = jnp.zeros((N, F), f32)
    dstg_all = jnp.concatenate([dst, dst + N])
    srcg_all = jnp.concatenate([src, src + N])

    pd1, ps1 = _proj1(x.astype(f32), wpd1, wps1)
    part1 = _sc_edge(pd1, ps1, c1, dstg_all, srcg_all, dst, zeros_n)

    h1, pd2, ps2 = _proj2(x.astype(f32), part1, wpd2, wps2)
    part2 = _sc_edge(pd2, ps2, c2, dstg_all, srcg_all, dst, zeros_n)

    batch3d = batch.astype(jnp.int32).reshape(NBLK, 1, BN)
    return _tail(h1, part2, batch3d, W1, b1.reshape(1, H), W2,
                 b2.reshape(1, H), Wout.reshape(1, H), bout.reshape(1, 1))


# speedup vs baseline: 1.1564x; 1.1564x over previous
import jax
import jax.numpy as jnp
from jax import lax
from jax.experimental import pallas as pl
from jax.experimental.pallas import tpu as pltpu
from jax.experimental.pallas import tpu_sc as plsc

N = 10000
E = 320000
F = 128
DE = 16
H = 64
G = 128

NCORES = 2
NSUB = 16
FH = F // 2
ECT = E // NSUB
CH = 80
IB = 2000
IBC = IB // CH
NBI = ECT // IB
RPT = 624
NREM = N - NSUB * RPT

BN = 400
NBLK = N // BN
BE = 2000
EBLK = E // BE


def _lrelu(v):
    return jnp.where(v >= 0, v, 0.01 * v)


def _edgefeat_body(ea_ref, w_ref, b_ref, c_ref):
    c_ref[...] = jnp.dot(ea_ref[...], w_ref[...],
                         preferred_element_type=jnp.float32) + b_ref[...]


def _edge_features(edge_attr, w_e, b_e):
    return pl.pallas_call(
        _edgefeat_body,
        grid=(EBLK, NCORES),
        in_specs=[
            pl.BlockSpec((BE, DE), lambda i, g: (i, 0)),
            pl.BlockSpec((DE, F), lambda i, g: (0, g)),
            pl.BlockSpec((1, F), lambda i, g: (0, g)),
        ],
        out_specs=pl.BlockSpec((BE, F), lambda i, g: (g * EBLK + i, 0)),
        out_shape=jax.ShapeDtypeStruct((NCORES * E, F), jnp.float32),
    )(edge_attr, w_e, b_e)


def _proj1_body(x_ref, wpd_ref, wps_ref, pd_ref, ps_ref):
    h = x_ref[...]
    pd_ref[...] = jnp.dot(h, wpd_ref[...], preferred_element_type=jnp.float32)
    ps_ref[...] = jnp.dot(h, wps_ref[...], preferred_element_type=jnp.float32)


def _proj1(x, wpd, wps):
    return pl.pallas_call(
        _proj1_body,
        grid=(NBLK, NCORES),
        in_specs=[
            pl.BlockSpec((BN, F), lambda i, g: (i, 0)),
            pl.BlockSpec((F, F), lambda i, g: (0, g)),
            pl.BlockSpec((F, F), lambda i, g: (0, g)),
        ],
        out_specs=[
            pl.BlockSpec((BN, F), lambda i, g: (g * NBLK + i, 0)),
            pl.BlockSpec((BN, F), lambda i, g: (g * NBLK + i, 0)),
        ],
        out_shape=[
            jax.ShapeDtypeStruct((NCORES * N, F), jnp.float32),
            jax.ShapeDtypeStruct((NCORES * N, F), jnp.float32),
        ],
    )(x, wpd, wps)


def _proj2_body(x_ref, p0_ref, p1_ref, wpd_ref, wps_ref, h_ref, pd_ref, ps_ref):
    h = x_ref[...] + jnp.concatenate([p0_ref[..., :FH], p1_ref[..., :FH]],
                                     axis=1)
    h_ref[...] = h
    pd_ref[...] = jnp.dot(h, wpd_ref[...], preferred_element_type=jnp.float32)
    ps_ref[...] = jnp.dot(h, wps_ref[...], preferred_element_type=jnp.float32)


def _proj2(x, partials, wpd, wps):
    return pl.pallas_call(
        _proj2_body,
        grid=(NBLK, NCORES),
        in_specs=[
            pl.BlockSpec((BN, F), lambda i, g: (i, 0)),
            pl.BlockSpec((BN, F), lambda i, g: (i, 0)),
            pl.BlockSpec((BN, F), lambda i, g: (i + NBLK, 0)),
            pl.BlockSpec((F, F), lambda i, g: (0, g)),
            pl.BlockSpec((F, F), lambda i, g: (0, g)),
        ],
        out_specs=[
            pl.BlockSpec((BN, F), lambda i, g: (i, 0)),
            pl.BlockSpec((BN, F), lambda i, g: (g * NBLK + i, 0)),
            pl.BlockSpec((BN, F), lambda i, g: (g * NBLK + i, 0)),
        ],
        out_shape=[
            jax.ShapeDtypeStruct((N, F), jnp.float32),
            jax.ShapeDtypeStruct((NCORES * N, F), jnp.float32),
            jax.ShapeDtypeStruct((NCORES * N, F), jnp.float32),
        ],
    )(x, partials, partials, wpd, wps)


_L2E = 1.4426950408889634
_MAGIC = 12582912.0


def _exp2_16(y):
    t = y + _MAGIC
    f = y - (t - _MAGIC)
    kbits = lax.bitcast_convert_type(t, jnp.int32) - 0x4B3FFF81
    scale = lax.bitcast_convert_type(kbits << 23, jnp.float32)
    p = 0.99992448 + f * (0.69312103 + f * (0.24264008 + f * 0.055922036))
    return scale * p


def _rcp16(b):
    r = lax.bitcast_convert_type(0x7EF311C3 - lax.bitcast_convert_type(b, jnp.int32),
                                 jnp.float32)
    r = r * (2.0 - b * r)
    r = r * (2.0 - b * r)
    return r


def _log1p16(w):
    return 9.9750326e-06 + w * (0.99923548 + w * (-0.49023072 + w * (0.28527268
        + w * (-0.13158183 + w * 0.030449005))))


def _gated_msg16(uf, us):
    yf = jnp.maximum(jnp.minimum(uf * -_L2E, 126.0), -126.0)
    ef = _exp2_16(yf)
    g = _rcp16(1.0 + ef)
    ys = jnp.maximum(jnp.abs(us) * -_L2E, -126.0)
    w = _exp2_16(ys)
    sp = jnp.maximum(us, 0.0) + _log1p16(w)
    return g * sp


def _sc_edge_body(pd_hbm, ps_hbm, c_hbm, dstg_hbm, srcg_hbm, dstr_hbm,
                  zeros_hbm, out_hbm,
                  dstr_v, dstg_v, srcg_v, dstc_v, pd_v, ps_v, c_v, msg_v,
                  acc_sh, sem):
    cid = lax.axis_index("c")
    sid = lax.axis_index("s")
    noff = cid * N
    ebase = sid * ECT
    goff = cid * E + ebase
    rbase = sid * RPT
    xbase = NSUB * RPT + sid * 8

    pltpu.sync_copy(zeros_hbm.at[pl.ds(rbase, RPT)],
                    acc_sh.at[pl.ds(rbase, RPT)])

    @pl.when(sid < NREM // 8)
    def _():
        pltpu.sync_copy(zeros_hbm.at[pl.ds(xbase, 8)],
                        acc_sh.at[pl.ds(xbase, 8)])

    def zm(e, cc):
        for jj in range(FH // 16, F // 16):
            msg_v[e, pl.ds(16 * jj, 16)] = jnp.zeros((16,), jnp.float32)
        return cc

    lax.fori_loop(0, CH, zm, 0)
    plsc.subcore_barrier()

    def block(b, carry):
        off = ebase + b * IB
        go = goff + b * IB
        pltpu.sync_copy(dstr_hbm.at[pl.ds(off, IB)], dstr_v)
        pltpu.sync_copy(dstg_hbm.at[pl.ds(go, IB)], dstg_v)
        pltpu.sync_copy(srcg_hbm.at[pl.ds(go, IB)], srcg_v)

        def chunk(j, cc):
            d1 = pltpu.async_copy(pd_hbm.at[dstg_v.at[pl.ds(j * CH, CH)]],
                                  pd_v, sem)
            d2 = pltpu.async_copy(ps_hbm.at[srcg_v.at[pl.ds(j * CH, CH)]],
                                  ps_v, sem)
            d3 = pltpu.async_copy(c_hbm.at[pl.ds(go + j * CH, CH)], c_v, sem)
            for q in range(CH // 16):
                dstc_v[pl.ds(q * 16, 16)] = dstr_v[pl.ds(j * CH + q * 16, 16)]
            d1.wait()
            d2.wait()
            d3.wait()

            @plsc.parallel_loop(0, CH, 1, unroll=4)
            def edge(e):
                for jj in range(FH // 16):
                    lo = 16 * jj
                    uf = pd_v[e, pl.ds(lo, 16)] + ps_v[e, pl.ds(lo, 16)] \
                        + c_v[e, pl.ds(lo, 16)]
                    us = pd_v[e, pl.ds(FH + lo, 16)] \
                        + ps_v[e, pl.ds(FH + lo, 16)] \
                        + c_v[e, pl.ds(FH + lo, 16)]
                    msg_v[e, pl.ds(lo, 16)] = _gated_msg16(uf, us)

            pltpu.sync_copy(msg_v, acc_sh.at[dstc_v], add=True)
            return cc

        lax.fori_loop(0, IBC, chunk, 0)
        return carry

    lax.fori_loop(0, NBI, block, 0)
    plsc.subcore_barrier()
    pltpu.sync_copy(acc_sh.at[pl.ds(rbase, RPT)],
                    out_hbm.at[pl.ds(noff + rbase, RPT)])

    @pl.when(sid < NREM // 8)
    def _():
        pltpu.sync_copy(acc_sh.at[pl.ds(xbase, 8)],
                        out_hbm.at[pl.ds(noff + xbase, 8)])


_sc_edge = pl.kernel(
    _sc_edge_body,
    out_type=jax.ShapeDtypeStruct((NCORES * N, F), jnp.float32),
    mesh=plsc.VectorSubcoreMesh(core_axis_name="c", subcore_axis_name="s"),
    scratch_types=(
        [pltpu.VMEM((IB,), jnp.int32)] * 3
        + [pltpu.VMEM((CH,), jnp.int32)]
        + [pltpu.VMEM((CH, F), jnp.float32)] * 4
        + [
            pltpu.VMEM_SHARED((N, F), jnp.float32),
            pltpu.SemaphoreType.DMA,
        ]
    ),
)


def _tail_body(h1_ref, p0_ref, p1_ref, batch_ref, w1_ref, b1_ref, w2_ref,
               b2_ref, wout_ref, bout_ref, out_ref, gacc):
    i = pl.program_id(0)

    @pl.when(i == 0)
    def _():
        gacc[...] = jnp.zeros_like(gacc)

    h2 = h1_ref[...] + jnp.concatenate([p0_ref[..., :FH], p1_ref[..., :FH]],
                                       axis=1)
    a = _lrelu(h2)
    t = jnp.dot(a, w1_ref[...], preferred_element_type=jnp.float32)
    t = _lrelu(t + b1_ref[...])
    gid = batch_ref[0, 0, :]
    cols = lax.broadcasted_iota(jnp.int32, (BN, G), 1)
    oh = (gid[:, None] == cols).astype(jnp.float32)
    gacc[...] += lax.dot_general(oh, t, (((0,), (0,)), ((), ())),
                                 preferred_element_type=jnp.float32)

    @pl.when(i == NBLK - 1)
    def _():
        g = gacc[...]
        z = jnp.dot(g, w2_ref[...], preferred_element_type=jnp.float32)
        z = _lrelu(z + b2_ref[...])
        o = jnp.sum(z * wout_ref[...], axis=1, keepdims=True) + bout_ref[...]
        out_ref[...] = jnp.maximum(o, 0.0)


def _tail(h1, partials, batch3d, w1, b1, w2, b2, wout_row, bout11):
    return pl.pallas_call(
        _tail_body,
        grid=(NBLK,),
        in_specs=[
            pl.BlockSpec((BN, F), lambda i: (i, 0)),
            pl.BlockSpec((BN, F), lambda i: (i, 0)),
            pl.BlockSpec((BN, F), lambda i: (i + NBLK, 0)),
            pl.BlockSpec((1, 1, BN), lambda i: (i, 0, 0)),
            pl.BlockSpec((F, H), lambda i: (0, 0)),
            pl.BlockSpec((1, H), lambda i: (0, 0)),
            pl.BlockSpec((H, H), lambda i: (0, 0)),
            pl.BlockSpec((1, H), lambda i: (0, 0)),
            pl.BlockSpec((1, H), lambda i: (0, 0)),
            pl.BlockSpec((1, 1), lambda i: (0, 0)),
        ],
        out_specs=pl.BlockSpec((G, 1), lambda i: (0, 0)),
        out_shape=jax.ShapeDtypeStruct((G, 1), jnp.float32),
        scratch_shapes=[pltpu.VMEM((G, H), jnp.float32)],
    )(h1, partials, partials, batch3d, w1, b1, w2, b2, wout_row, bout11)


def kernel(x, edge_attr, Wf1, bf1, Ws1, bs1, Wf2, bf2, Ws2, bs2,
           W1, b1, W2, b2, Wout, bout, edge_index, batch):
    f32 = jnp.float32
    src = edge_index[0].astype(jnp.int32)
    dst = edge_index[1].astype(jnp.int32)

    def corecat(wf, ws):
        return jnp.concatenate([wf[:, :FH], ws[:, :FH], wf[:, FH:], ws[:, FH:]], 1)

    wpd1 = corecat(Wf1[:F], Ws1[:F])
    wps1 = corecat(Wf1[F:2 * F], Ws1[F:2 * F])
    wpd2 = corecat(Wf2[:F], Ws2[:F])
    wps2 = corecat(Wf2[F:2 * F], Ws2[F:2 * F])
    w_e1 = corecat(Wf1[2 * F:], Ws1[2 * F:])
    w_e2 = corecat(Wf2[2 * F:], Ws2[2 * F:])
    b_e1 = corecat(bf1.reshape(1, F), bs1.reshape(1, F))
    b_e2 = corecat(bf2.reshape(1, F), bs2.reshape(1, F))

    c1 = _edge_features(edge_attr.astype(f32), w_e1, b_e1)
    zeros_n = jnp.zeros((N, F), f32)
    dstg_all = jnp.concatenate([dst, dst + N])
    srcg_all = jnp.concatenate([src, src + N])

    pd1, ps1 = _proj1(x.astype(f32), wpd1, wps1)
    part1 = _sc_edge(pd1, ps1, c1, dstg_all, srcg_all, dst, zeros_n)

    c2 = _edge_features(edge_attr.astype(f32), w_e2, b_e2)
    h1, pd2, ps2 = _proj2(x.astype(f32), part1, wpd2, wps2)
    part2 = _sc_edge(pd2, ps2, c2, dstg_all, srcg_all, dst, zeros_n)

    batch3d = batch.astype(jnp.int32).reshape(NBLK, 1, BN)
    return _tail(h1, part2, batch3d, W1, b1.reshape(1, H), W2,
                 b2.reshape(1, H), Wout.reshape(1, H), bout.reshape(1, 1))
